# baseline (speedup 1.0000x reference)
import functools

import jax
import jax.numpy as jnp
import numpy as np
from jax import lax
from jax.experimental import pallas as pl
from jax.experimental.pallas import tpu as pltpu
from jax.experimental.pallas import tpu_sc as plsc

N_NODES = 50000
N_EDGES = 800000
XDIM = 16
HID = 32
N_P = 8
TRI = 36
FDIM = N_P + TRI
PAY = 48
YDIM = 112
EPS = 1e-7

BE = 1600
BN = 2000

_PERM = np.array([8 + j if j < 28 else 63 - j for j in range(64)], np.int32)
_P36 = np.zeros((TRI, 64), np.float32)
for _j in range(64):
    _P36[_PERM[_j], _j] = 1.0
_D36 = np.zeros((TRI, N_P), np.float32)
for _i in range(N_P):
    _D36[_PERM[9 * _i], _i] = 1.0
_DEX = np.zeros((N_P, 64), np.float32)
for _i in range(N_P):
    _DEX[_i, 9 * _i] = 1.0

_PAIR = {}
_cnt = 0
for _i in range(N_P):
    for _k in range(_i, N_P):
        _PAIR[(_i, _k)] = _cnt
        _cnt += 1

_C512 = np.zeros((512, TRI), np.float32)
for _d in range(N_P):
    for _i in range(N_P - _d):
        for _j in range(N_P):
            _C512[64 * _d + 8 * _i + _j, _PAIR[(_i, _i + _d)]] = 1.0

_TRI_AT = [[_PAIR[(min(_i, _k), max(_i, _k))] for _k in range(N_P)]
           for _i in range(N_P)]


def _softplus(x):
    return jnp.log1p(jnp.exp(-jnp.abs(x))) + jnp.maximum(x, 0.0)


def _layer_norm(h, g, b):
    mu = jnp.mean(h, axis=-1, keepdims=True)
    var = jnp.mean((h - mu) ** 2, axis=-1, keepdims=True)
    return (h - mu) / jnp.sqrt(var + 1e-5) * g + b


def _embed_body(x_ref, w_ref, b_ref, o_ref):
    o_ref[...] = x_ref[...] @ w_ref[...] + b_ref[...]


def _embed(x, w, b):
    n, d = x.shape
    return pl.pallas_call(
        _embed_body,
        grid=(n // BN,),
        in_specs=[
            pl.BlockSpec((BN, d), lambda i: (i, 0)),
            pl.BlockSpec((d, w.shape[1]), lambda i: (0, 0)),
            pl.BlockSpec((1, w.shape[1]), lambda i: (0, 0)),
        ],
        out_specs=pl.BlockSpec((BN, w.shape[1]), lambda i: (i, 0)),
        out_shape=jax.ShapeDtypeStruct((n, w.shape[1]), jnp.float32),
    )(x, w, b.reshape(1, -1))


def _edge_body(g_ref, ea_ref, ew_ref, eb_ref, a1w_ref, a1b_ref,
               p36_ref, d36_ref, dex_ref, c512_ref, o_ref):
    g = g_ref[...]
    e_emb = ea_ref[...] @ ew_ref[...] + eb_ref[...]
    msg = jnp.maximum(g + e_emb, 0.0) + EPS
    h = msg @ a1w_ref[...] + a1b_ref[...]
    score = h[:, :N_P]
    fisher = h[:, N_P:]
    qf = fisher @ p36_ref[...]
    diag = fisher @ d36_ref[...]
    middle = diag - _softplus(diag)
    l_flat = qf - middle @ dex_ref[...]
    prods = [l_flat * l_flat]
    for d in range(1, N_P):
        rolled = jnp.concatenate(
            [l_flat[:, 8 * d:], l_flat[:, :8 * d]], axis=1)
        prods.append(l_flat * rolled)
    prod = jnp.concatenate(prods, axis=1)
    ftri = prod @ c512_ref[...]
    o_ref[:, :N_P] = score
    o_ref[:, N_P:FDIM] = ftri
    o_ref[:, FDIM:] = jnp.zeros((score.shape[0], PAY - FDIM), jnp.float32)


def _edge_stage(g, edge_attr, ew, eb, a1w, a1b):
    e = g.shape[0]
    return pl.pallas_call(
        _edge_body,
        grid=(e // BE,),
        in_specs=[
            pl.BlockSpec((BE, HID), lambda i: (i, 0)),
            pl.BlockSpec((BE, XDIM), lambda i: (i, 0)),
            pl.BlockSpec((XDIM, HID), lambda i: (0, 0)),
            pl.BlockSpec((1, HID), lambda i: (0, 0)),
            pl.BlockSpec((HID, FDIM), lambda i: (0, 0)),
            pl.BlockSpec((1, FDIM), lambda i: (0, 0)),
            pl.BlockSpec((TRI, 64), lambda i: (0, 0)),
            pl.BlockSpec((TRI, N_P), lambda i: (0, 0)),
            pl.BlockSpec((N_P, 64), lambda i: (0, 0)),
            pl.BlockSpec((512, TRI), lambda i: (0, 0)),
        ],
        out_specs=pl.BlockSpec((BE, PAY), lambda i: (i, 0)),
        out_shape=jax.ShapeDtypeStruct((e, PAY), jnp.float32),
    )(g, edge_attr, ew, eb.reshape(1, -1), a1w, a1b.reshape(1, -1),
      jnp.asarray(_P36), jnp.asarray(_D36), jnp.asarray(_DEX),
      jnp.asarray(_C512))


def _solve(acc):
    s = acc[:, :N_P]
    rows = []
    for i in range(N_P):
        cols = []
        for k in range(N_P):
            c = acc[:, N_P + _TRI_AT[i][k]:N_P + _TRI_AT[i][k] + 1]
            if k == i:
                c = c + 1.0
            cols.append(c)
        rows.append(jnp.concatenate(cols, axis=1))
    lane = lax.broadcasted_iota(jnp.int32, (1, N_P), 1)
    for k in range(N_P):
        ip = 1.0 / rows[k][:, k:k + 1]
        rowk = rows[k] * ip
        sk = s[:, k:k + 1] * ip
        cs = [rows[i][:, k:k + 1] for i in range(N_P)]
        new_rows = []
        for i in range(N_P):
            if i == k:
                new_rows.append(rowk)
            else:
                new_rows.append(rows[i] - cs[i] * rowk)
        rows = new_rows
        ck0 = jnp.concatenate(
            [jnp.zeros_like(cs[0]) if i == k else cs[i] for i in range(N_P)],
            axis=1)
        s = jnp.where(lane == k, sk, s - ck0 * sk)
    return s


def _node0_body(acc_ref, x0_ref, a2w_ref, a2b_ref, m1w_ref, m1b_ref,
                mln_ref, m2w_ref, m2b_ref, ln1_ref, x1_ref, t1_ref):
    mle = _solve(acc_ref[...])
    out = mle @ a2w_ref[...] + a2b_ref[...] + x0_ref[...]
    h = out @ m1w_ref[...] + m1b_ref[...]
    h = _layer_norm(h, mln_ref[0:1, :], mln_ref[1:2, :])
    h = jnp.maximum(h, 0.0)
    x1 = h @ m2w_ref[...] + m2b_ref[...]
    t1 = jnp.maximum(_layer_norm(x1, ln1_ref[0:1, :], ln1_ref[1:2, :]), 0.0)
    x1_ref[...] = x1
    t1_ref[...] = t1


def _node_stage0(acc, x0, p):
    n = acc.shape[0]
    mln = jnp.stack([p['mlnW_0'], p['mlnB_0']])
    ln1 = jnp.stack([p['lnW_1'], p['lnB_1']])
    return pl.pallas_call(
        _node0_body,
        grid=(n // BN,),
        in_specs=[
            pl.BlockSpec((BN, PAY), lambda i: (i, 0)),
            pl.BlockSpec((BN, HID), lambda i: (i, 0)),
            pl.BlockSpec((N_P, HID), lambda i: (0, 0)),
            pl.BlockSpec((1, HID), lambda i: (0, 0)),
            pl.BlockSpec((HID, 2 * HID), lambda i: (0, 0)),
            pl.BlockSpec((1, 2 * HID), lambda i: (0, 0)),
            pl.BlockSpec((2, 2 * HID), lambda i: (0, 0)),
            pl.BlockSpec((2 * HID, HID), lambda i: (0, 0)),
            pl.BlockSpec((1, HID), lambda i: (0, 0)),
            pl.BlockSpec((2, HID), lambda i: (0, 0)),
        ],
        out_specs=[
            pl.BlockSpec((BN, HID), lambda i: (i, 0)),
            pl.BlockSpec((BN, HID), lambda i: (i, 0)),
        ],
        out_shape=[
            jax.ShapeDtypeStruct((n, HID), jnp.float32),
            jax.ShapeDtypeStruct((n, HID), jnp.float32),
        ],
    )(acc, x0, p['a2W_0'], p['a2b_0'].reshape(1, -1), p['m1W_0'],
      p['m1b_0'].reshape(1, -1), mln, p['m2W_0'], p['m2b_0'].reshape(1, -1),
      ln1)


def _node1_body(acc_ref, t1_ref, x1_ref, a2w_ref, a2b_ref, m1w_ref, m1b_ref,
                mln_ref, m2w_ref, m2b_ref, ln0_ref, linw_ref, linb_ref,
                y_ref):
    mle = _solve(acc_ref[...])
    out = mle @ a2w_ref[...] + a2b_ref[...] + t1_ref[...]
    h = out @ m1w_ref[...] + m1b_ref[...]
    h = _layer_norm(h, mln_ref[0:1, :], mln_ref[1:2, :])
    h = jnp.maximum(h, 0.0)
    h = h @ m2w_ref[...] + m2b_ref[...]
    x2 = x1_ref[...] + h
    z = jnp.maximum(_layer_norm(x2, ln0_ref[0:1, :], ln0_ref[1:2, :]), 0.0)
    y_ref[...] = z @ linw_ref[...] + linb_ref[...]


def _node_stage1(acc, t1, x1, p):
    n = acc.shape[0]
    mln = jnp.stack([p['mlnW_1'], p['mlnB_1']])
    ln0 = jnp.stack([p['lnW_0'], p['lnB_0']])
    return pl.pallas_call(
        _node1_body,
        grid=(n // BN,),
        in_specs=[
            pl.BlockSpec((BN, PAY), lambda i: (i, 0)),
            pl.BlockSpec((BN, HID), lambda i: (i, 0)),
            pl.BlockSpec((BN, HID), lambda i: (i, 0)),
            pl.BlockSpec((N_P, HID), lambda i: (0, 0)),
            pl.BlockSpec((1, HID), lambda i: (0, 0)),
            pl.BlockSpec((HID, 2 * HID), lambda i: (0, 0)),
            pl.BlockSpec((1, 2 * HID), lambda i: (0, 0)),
            pl.BlockSpec((2, 2 * HID), lambda i: (0, 0)),
            pl.BlockSpec((2 * HID, HID), lambda i: (0, 0)),
            pl.BlockSpec((1, HID), lambda i: (0, 0)),
            pl.BlockSpec((2, HID), lambda i: (0, 0)),
            pl.BlockSpec((HID, YDIM), lambda i: (0, 0)),
            pl.BlockSpec((1, YDIM), lambda i: (0, 0)),
        ],
        out_specs=pl.BlockSpec((BN, YDIM), lambda i: (i, 0)),
        out_shape=jax.ShapeDtypeStruct((n, YDIM), jnp.float32),
    )(acc, t1, x1, p['a2W_1'], p['a2b_1'].reshape(1, -1), p['m1W_1'],
      p['m1b_1'].reshape(1, -1), mln, p['m2W_1'], p['m2b_1'].reshape(1, -1),
      ln0, p['lin_W'], p['lin_b'].reshape(1, -1))


_NW = 32
_CH = 128
_GRP = 5
_CPW = 195
_NGRP = _CPW // _GRP


def _sc_gather(table, idx):
    e = idx.shape[0]
    n_chunks = e // _CH
    d = table.shape[1]
    mesh = plsc.VectorSubcoreMesh(core_axis_name="c", subcore_axis_name="s")

    @functools.partial(
        pl.kernel,
        out_type=jax.ShapeDtypeStruct((e, d), jnp.float32),
        mesh=mesh,
        scratch_types=[
            pltpu.VMEM((_GRP * _CH,), jnp.int32),
            pltpu.VMEM((_GRP * _CH, d), jnp.float32),
            pltpu.VMEM_SHARED((N_NODES, HID), jnp.float32),
            pltpu.SemaphoreType.DMA,
        ],
        compiler_params=pltpu.CompilerParams(use_tc_tiling_on_sc=False),
    )
    def k(table_h, idx_h, out_h, idx_v, rows_v, table_s, sem):
        wid = lax.axis_index("s") * 2 + lax.axis_index("c")
        sub = lax.axis_index("s")
        base_chunk = wid * _CPW

        @pl.when(sub == 0)
        def _stage():
            pltpu.sync_copy(table_h, table_s)

        plsc.subcore_barrier()

        def group(g, _):
            c0 = base_chunk + g * _GRP
            pltpu.sync_copy(idx_h.at[pl.ds(c0 * _CH, _GRP * _CH)], idx_v)
            cps = [
                pltpu.async_copy(
                    table_s.at[idx_v.at[pl.ds(j * _CH, _CH)]],
                    rows_v.at[pl.ds(j * _CH, _CH)], sem)
                for j in range(_GRP)
            ]
            for cp in cps:
                cp.wait()
            pltpu.sync_copy(rows_v, out_h.at[pl.ds(c0 * _CH, _GRP * _CH)])
            return _

        lax.fori_loop(0, _NGRP, group, 0)

        @pl.when(wid < n_chunks - _NW * _CPW)
        def _extra():
            c0 = _NW * _CPW + wid
            pltpu.sync_copy(idx_h.at[pl.ds(c0 * _CH, _CH)],
                            idx_v.at[pl.ds(0, _CH)])
            pltpu.async_copy(
                table_s.at[idx_v.at[pl.ds(0, _CH)]],
                rows_v.at[pl.ds(0, _CH)], sem).wait()
            pltpu.sync_copy(rows_v.at[pl.ds(0, _CH)],
                            out_h.at[pl.ds(c0 * _CH, _CH)])

    return k(table, idx)


def _gather(table, idx):
    return _sc_gather(table, idx)


_HALF = N_NODES // 2
_ACC_R = _HALF + 8
_CPT = 390
_ZR = 128


def _sc_scatter(pay, dst, zeros):
    e = pay.shape[0]
    n_chunks = e // _CH
    mesh = plsc.VectorSubcoreMesh(core_axis_name="c", subcore_axis_name="s")

    @functools.partial(
        pl.kernel,
        out_type=jax.ShapeDtypeStruct((N_NODES, PAY), jnp.float32),
        mesh=mesh,
        scratch_types=[
            pltpu.VMEM((_CH, PAY), jnp.float32),
            pltpu.VMEM((_CH,), jnp.int32),
            pltpu.VMEM((_CH,), jnp.int32),
            pltpu.VMEM_SHARED((_ACC_R, PAY), jnp.float32),
        ],
        compiler_params=pltpu.CompilerParams(use_tc_tiling_on_sc=False),
    )
    def k(pay_h, dst_h, zero_h, out_h, pay_v, dst_v, loc_v, acc_s):
        core = lax.axis_index("c")
        sub = lax.axis_index("s")
        base = core * _HALF

        rpt = _ACC_R // 16

        def zacc(i, _):
            pltpu.sync_copy(zero_h,
                            acc_s.at[pl.ds(sub * rpt + i * _ZR, _ZR)])
            return _

        nfull = rpt // _ZR
        rem = rpt - nfull * _ZR
        lax.fori_loop(0, nfull, zacc, 0)
        pltpu.sync_copy(zero_h.at[pl.ds(0, rem)],
                        acc_s.at[pl.ds(sub * rpt + nfull * _ZR, rem)])

        plsc.subcore_barrier()

        def chunk(cid):
            e0 = cid * _CH
            pltpu.sync_copy(pay_h.at[pl.ds(e0, _CH)], pay_v)
            pltpu.sync_copy(dst_h.at[pl.ds(e0, _CH)], dst_v)
            for i in range(_CH // 16):
                v = dst_v[pl.ds(i * 16, 16)] - base
                ok = (v >= 0) & (v < _HALF)
                loc_v[pl.ds(i * 16, 16)] = jnp.where(
                    ok, v, jnp.full((16,), _HALF, jnp.int32))
            pltpu.sync_copy(pay_v, acc_s.at[loc_v], add=True)

        def main(i, _):
            chunk(sub * _CPT + i)
            return _

        lax.fori_loop(0, _CPT, main, 0)

        @pl.when(sub < n_chunks - 16 * _CPT)
        def _extra():
            chunk(16 * _CPT + sub)

        plsc.subcore_barrier()

        wrows = _HALF // 16

        def wout(i, _):
            r0 = sub * wrows + i * _ZR
            pltpu.sync_copy(acc_s.at[pl.ds(r0, _ZR)],
                            out_h.at[pl.ds(base + r0, _ZR)])
            return _

        nw = wrows // _ZR
        wrem = wrows - nw * _ZR
        lax.fori_loop(0, nw, wout, 0)
        pltpu.sync_copy(acc_s.at[pl.ds(sub * wrows + nw * _ZR, wrem)],
                        out_h.at[pl.ds(base + sub * wrows + nw * _ZR, wrem)])

        @pl.when(sub == 0)
        def _last8():
            pltpu.sync_copy(acc_s.at[pl.ds(16 * wrows, 8)],
                            out_h.at[pl.ds(base + 16 * wrows, 8)])

    return k(pay, dst, zeros)


def _scatter_sum(payload, dst, zeros):
    return _sc_scatter(payload, dst, zeros)


def kernel(x, edge_index, edge_attr, params):
    p = params
    src = edge_index[0]
    dst = edge_index[1]
    zeros = jnp.zeros((_ZR, PAY), jnp.float32)
    x0 = _embed(x, p['node_W'], p['node_b'])

    g0 = _gather(x0, src)
    r0 = _edge_stage(g0, edge_attr, p['edge_W'], p['edge_b'],
                     p['a1W_0'], p['a1b_0'])
    acc0 = _scatter_sum(r0, dst, zeros)
    x1, t1 = _node_stage0(acc0, x0, p)

    g1 = _gather(t1, src)
    r1 = _edge_stage(g1, edge_attr, p['edge_W'], p['edge_b'],
                     p['a1W_1'], p['a1b_1'])
    acc1 = _scatter_sum(r1, dst, zeros)
    return _node_stage1(acc1, t1, x1, p)


# speedup vs baseline: 1.3390x; 1.3390x over previous
import functools

import jax
import jax.numpy as jnp
import numpy as np
from jax import lax
from jax.experimental import pallas as pl
from jax.experimental.pallas import tpu as pltpu
from jax.experimental.pallas import tpu_sc as plsc

N_NODES = 50000
N_EDGES = 800000
XDIM = 16
HID = 32
N_P = 8
TRI = 36
FDIM = N_P + TRI
PAY = 48
YDIM = 112
EPS = 1e-7

BE = 1600
BN = 2000

_PERM = np.array([8 + j if j < 28 else 63 - j for j in range(64)], np.int32)
_P36 = np.zeros((TRI, 64), np.float32)
for _j in range(64):
    _P36[_PERM[_j], _j] = 1.0
_D36 = np.zeros((TRI, N_P), np.float32)
for _i in range(N_P):
    _D36[_PERM[9 * _i], _i] = 1.0
_DEX = np.zeros((N_P, 64), np.float32)
for _i in range(N_P):
    _DEX[_i, 9 * _i] = 1.0

_PAIR = {}
_cnt = 0
for _i in range(N_P):
    for _k in range(_i, N_P):
        _PAIR[(_i, _k)] = _cnt
        _cnt += 1

_A = np.zeros((64, 512), np.float32)
_B = np.zeros((64, 512), np.float32)
for _j in range(N_P):
    for _i in range(N_P):
        for _k in range(N_P):
            _c = 64 * _j + 8 * _i + _k
            _A[8 * _i + _j, _c] = 1.0
            _B[8 * _k + _j, _c] = 1.0
_AB = np.concatenate([_A, _B], axis=1)
_S36 = np.zeros((512, TRI), np.float32)
for _j in range(N_P):
    for _i in range(N_P):
        for _k in range(_i, N_P):
            _S36[64 * _j + 8 * _i + _k, _PAIR[(_i, _k)]] = 1.0

_TRI_AT = [[_PAIR[(min(_i, _k), max(_i, _k))] for _k in range(N_P)]
           for _i in range(N_P)]


def _softplus(x):
    return jnp.log1p(jnp.exp(-jnp.abs(x))) + jnp.maximum(x, 0.0)


def _layer_norm(h, g, b):
    mu = jnp.mean(h, axis=-1, keepdims=True)
    var = jnp.mean((h - mu) ** 2, axis=-1, keepdims=True)
    return (h - mu) / jnp.sqrt(var + 1e-5) * g + b


def _embed_body(x_ref, w_ref, b_ref, o_ref):
    o_ref[...] = x_ref[...] @ w_ref[...] + b_ref[...]


def _embed(x, w, b):
    n, d = x.shape
    return pl.pallas_call(
        _embed_body,
        grid=(n // BN,),
        in_specs=[
            pl.BlockSpec((BN, d), lambda i: (i, 0)),
            pl.BlockSpec((d, w.shape[1]), lambda i: (0, 0)),
            pl.BlockSpec((1, w.shape[1]), lambda i: (0, 0)),
        ],
        out_specs=pl.BlockSpec((BN, w.shape[1]), lambda i: (i, 0)),
        out_shape=jax.ShapeDtypeStruct((n, w.shape[1]), jnp.float32),
    )(x, w, b.reshape(1, -1))


def _edge_body(g_ref, ea_ref, ew_ref, eb_ref, a1w_ref, a1b_ref,
               p36_ref, d36_ref, dex_ref, ab_ref, s36_ref, o_ref):
    g = g_ref[...]
    e_emb = ea_ref[...] @ ew_ref[...] + eb_ref[...]
    msg = jnp.maximum(g + e_emb, 0.0) + EPS
    h = msg @ a1w_ref[...] + a1b_ref[...]
    score = h[:, :N_P]
    fisher = h[:, N_P:]
    qf = fisher @ p36_ref[...]
    diag = fisher @ d36_ref[...]
    middle = diag - _softplus(diag)
    l_flat = qf - middle @ dex_ref[...]
    lab = l_flat @ ab_ref[...]
    prod = lab[:, :512] * lab[:, 512:]
    ftri = prod @ s36_ref[...]
    o_ref[:, :N_P] = score
    o_ref[:, N_P:FDIM] = ftri
    o_ref[:, FDIM:] = jnp.zeros((score.shape[0], PAY - FDIM), jnp.float32)


def _edge_stage(g, edge_attr, ew, eb, a1w, a1b):
    e = g.shape[0]
    return pl.pallas_call(
        _edge_body,
        grid=(e // BE,),
        in_specs=[
            pl.BlockSpec((BE, HID), lambda i: (i, 0)),
            pl.BlockSpec((BE, XDIM), lambda i: (i, 0)),
            pl.BlockSpec((XDIM, HID), lambda i: (0, 0)),
            pl.BlockSpec((1, HID), lambda i: (0, 0)),
            pl.BlockSpec((HID, FDIM), lambda i: (0, 0)),
            pl.BlockSpec((1, FDIM), lambda i: (0, 0)),
            pl.BlockSpec((TRI, 64), lambda i: (0, 0)),
            pl.BlockSpec((TRI, N_P), lambda i: (0, 0)),
            pl.BlockSpec((N_P, 64), lambda i: (0, 0)),
            pl.BlockSpec((64, 1024), lambda i: (0, 0)),
            pl.BlockSpec((512, TRI), lambda i: (0, 0)),
        ],
        out_specs=pl.BlockSpec((BE, PAY), lambda i: (i, 0)),
        out_shape=jax.ShapeDtypeStruct((e, PAY), jnp.float32),
    )(g, edge_attr, ew, eb.reshape(1, -1), a1w, a1b.reshape(1, -1),
      jnp.asarray(_P36), jnp.asarray(_D36), jnp.asarray(_DEX),
      jnp.asarray(_AB), jnp.asarray(_S36))


def _solve(acc):
    s = acc[:, :N_P]
    rows = []
    for i in range(N_P):
        cols = []
        for k in range(N_P):
            c = acc[:, N_P + _TRI_AT[i][k]:N_P + _TRI_AT[i][k] + 1]
            if k == i:
                c = c + 1.0
            cols.append(c)
        rows.append(jnp.concatenate(cols, axis=1))
    lane = lax.broadcasted_iota(jnp.int32, (1, N_P), 1)
    for k in range(N_P):
        ip = 1.0 / rows[k][:, k:k + 1]
        rowk = rows[k] * ip
        sk = s[:, k:k + 1] * ip
        cs = [rows[i][:, k:k + 1] for i in range(N_P)]
        new_rows = []
        for i in range(N_P):
            if i == k:
                new_rows.append(rowk)
            else:
                new_rows.append(rows[i] - cs[i] * rowk)
        rows = new_rows
        ck0 = jnp.concatenate(
            [jnp.zeros_like(cs[0]) if i == k else cs[i] for i in range(N_P)],
            axis=1)
        s = jnp.where(lane == k, sk, s - ck0 * sk)
    return s


def _node0_body(acc_ref, x0_ref, a2w_ref, a2b_ref, m1w_ref, m1b_ref,
                mln_ref, m2w_ref, m2b_ref, ln1_ref, x1_ref, t1_ref):
    mle = _solve(acc_ref[...])
    out = mle @ a2w_ref[...] + a2b_ref[...] + x0_ref[...]
    h = out @ m1w_ref[...] + m1b_ref[...]
    h = _layer_norm(h, mln_ref[0:1, :], mln_ref[1:2, :])
    h = jnp.maximum(h, 0.0)
    x1 = h @ m2w_ref[...] + m2b_ref[...]
    t1 = jnp.maximum(_layer_norm(x1, ln1_ref[0:1, :], ln1_ref[1:2, :]), 0.0)
    x1_ref[...] = x1
    t1_ref[...] = t1


def _node_stage0(acc, x0, p):
    n = acc.shape[0]
    mln = jnp.stack([p['mlnW_0'], p['mlnB_0']])
    ln1 = jnp.stack([p['lnW_1'], p['lnB_1']])
    return pl.pallas_call(
        _node0_body,
        grid=(n // BN,),
        in_specs=[
            pl.BlockSpec((BN, PAY), lambda i: (i, 0)),
            pl.BlockSpec((BN, HID), lambda i: (i, 0)),
            pl.BlockSpec((N_P, HID), lambda i: (0, 0)),
            pl.BlockSpec((1, HID), lambda i: (0, 0)),
            pl.BlockSpec((HID, 2 * HID), lambda i: (0, 0)),
            pl.BlockSpec((1, 2 * HID), lambda i: (0, 0)),
            pl.BlockSpec((2, 2 * HID), lambda i: (0, 0)),
            pl.BlockSpec((2 * HID, HID), lambda i: (0, 0)),
            pl.BlockSpec((1, HID), lambda i: (0, 0)),
            pl.BlockSpec((2, HID), lambda i: (0, 0)),
        ],
        out_specs=[
            pl.BlockSpec((BN, HID), lambda i: (i, 0)),
            pl.BlockSpec((BN, HID), lambda i: (i, 0)),
        ],
        out_shape=[
            jax.ShapeDtypeStruct((n, HID), jnp.float32),
            jax.ShapeDtypeStruct((n, HID), jnp.float32),
        ],
    )(acc, x0, p['a2W_0'], p['a2b_0'].reshape(1, -1), p['m1W_0'],
      p['m1b_0'].reshape(1, -1), mln, p['m2W_0'], p['m2b_0'].reshape(1, -1),
      ln1)


def _node1_body(acc_ref, t1_ref, x1_ref, a2w_ref, a2b_ref, m1w_ref, m1b_ref,
                mln_ref, m2w_ref, m2b_ref, ln0_ref, linw_ref, linb_ref,
                y_ref):
    mle = _solve(acc_ref[...])
    out = mle @ a2w_ref[...] + a2b_ref[...] + t1_ref[...]
    h = out @ m1w_ref[...] + m1b_ref[...]
    h = _layer_norm(h, mln_ref[0:1, :], mln_ref[1:2, :])
    h = jnp.maximum(h, 0.0)
    h = h @ m2w_ref[...] + m2b_ref[...]
    x2 = x1_ref[...] + h
    z = jnp.maximum(_layer_norm(x2, ln0_ref[0:1, :], ln0_ref[1:2, :]), 0.0)
    y_ref[...] = z @ linw_ref[...] + linb_ref[...]


def _node_stage1(acc, t1, x1, p):
    n = acc.shape[0]
    mln = jnp.stack([p['mlnW_1'], p['mlnB_1']])
    ln0 = jnp.stack([p['lnW_0'], p['lnB_0']])
    return pl.pallas_call(
        _node1_body,
        grid=(n // BN,),
        in_specs=[
            pl.BlockSpec((BN, PAY), lambda i: (i, 0)),
            pl.BlockSpec((BN, HID), lambda i: (i, 0)),
            pl.BlockSpec((BN, HID), lambda i: (i, 0)),
            pl.BlockSpec((N_P, HID), lambda i: (0, 0)),
            pl.BlockSpec((1, HID), lambda i: (0, 0)),
            pl.BlockSpec((HID, 2 * HID), lambda i: (0, 0)),
            pl.BlockSpec((1, 2 * HID), lambda i: (0, 0)),
            pl.BlockSpec((2, 2 * HID), lambda i: (0, 0)),
            pl.BlockSpec((2 * HID, HID), lambda i: (0, 0)),
            pl.BlockSpec((1, HID), lambda i: (0, 0)),
            pl.BlockSpec((2, HID), lambda i: (0, 0)),
            pl.BlockSpec((HID, YDIM), lambda i: (0, 0)),
            pl.BlockSpec((1, YDIM), lambda i: (0, 0)),
        ],
        out_specs=pl.BlockSpec((BN, YDIM), lambda i: (i, 0)),
        out_shape=jax.ShapeDtypeStruct((n, YDIM), jnp.float32),
    )(acc, t1, x1, p['a2W_1'], p['a2b_1'].reshape(1, -1), p['m1W_1'],
      p['m1b_1'].reshape(1, -1), mln, p['m2W_1'], p['m2b_1'].reshape(1, -1),
      ln0, p['lin_W'], p['lin_b'].reshape(1, -1))


_NW = 32
_CH = 128
_GRP = 5
_CPW = 195
_NGRP = _CPW // _GRP


def _sc_gather(table, idx):
    e = idx.shape[0]
    n_chunks = e // _CH
    d = table.shape[1]
    mesh = plsc.VectorSubcoreMesh(core_axis_name="c", subcore_axis_name="s")

    @functools.partial(
        pl.kernel,
        out_type=jax.ShapeDtypeStruct((e, d), jnp.float32),
        mesh=mesh,
        scratch_types=[
            pltpu.VMEM((_GRP * _CH,), jnp.int32),
            pltpu.VMEM((_GRP * _CH, d), jnp.float32),
            pltpu.VMEM_SHARED((N_NODES, HID), jnp.float32),
            pltpu.SemaphoreType.DMA,
        ],
        compiler_params=pltpu.CompilerParams(use_tc_tiling_on_sc=False),
    )
    def k(table_h, idx_h, out_h, idx_v, rows_v, table_s, sem):
        wid = lax.axis_index("s") * 2 + lax.axis_index("c")
        sub = lax.axis_index("s")
        base_chunk = wid * _CPW

        @pl.when(sub == 0)
        def _stage():
            pltpu.sync_copy(table_h, table_s)

        plsc.subcore_barrier()

        def group(g, _):
            c0 = base_chunk + g * _GRP
            pltpu.sync_copy(idx_h.at[pl.ds(c0 * _CH, _GRP * _CH)], idx_v)
            cps = [
                pltpu.async_copy(
                    table_s.at[idx_v.at[pl.ds(j * _CH, _CH)]],
                    rows_v.at[pl.ds(j * _CH, _CH)], sem)
                for j in range(_GRP)
            ]
            for cp in cps:
                cp.wait()
            pltpu.sync_copy(rows_v, out_h.at[pl.ds(c0 * _CH, _GRP * _CH)])
            return _

        lax.fori_loop(0, _NGRP, group, 0)

        @pl.when(wid < n_chunks - _NW * _CPW)
        def _extra():
            c0 = _NW * _CPW + wid
            pltpu.sync_copy(idx_h.at[pl.ds(c0 * _CH, _CH)],
                            idx_v.at[pl.ds(0, _CH)])
            pltpu.async_copy(
                table_s.at[idx_v.at[pl.ds(0, _CH)]],
                rows_v.at[pl.ds(0, _CH)], sem).wait()
            pltpu.sync_copy(rows_v.at[pl.ds(0, _CH)],
                            out_h.at[pl.ds(c0 * _CH, _CH)])

    return k(table, idx)


def _gather(table, idx):
    return _sc_gather(table, idx)


_HALF = N_NODES // 2
_ACC_R = _HALF + 8
_CPT = 390
_ZR = 128


def _sc_scatter(pay, dst, zeros):
    e = pay.shape[0]
    n_chunks = e // _CH
    mesh = plsc.VectorSubcoreMesh(core_axis_name="c", subcore_axis_name="s")

    @functools.partial(
        pl.kernel,
        out_type=jax.ShapeDtypeStruct((N_NODES, PAY), jnp.float32),
        mesh=mesh,
        scratch_types=[
            pltpu.VMEM((2, _CH, PAY), jnp.float32),
            pltpu.VMEM((2, _CH), jnp.int32),
            pltpu.VMEM((_CH,), jnp.int32),
            pltpu.VMEM_SHARED((_ACC_R, PAY), jnp.float32),
            pltpu.SemaphoreType.DMA,
        ],
        compiler_params=pltpu.CompilerParams(use_tc_tiling_on_sc=False),
    )
    def k(pay_h, dst_h, zero_h, out_h, pay_v, dst_v, loc_v, acc_s, sem):
        core = lax.axis_index("c")
        sub = lax.axis_index("s")
        base = core * _HALF

        rpt = _ACC_R // 16

        def zacc(i, _):
            pltpu.sync_copy(zero_h,
                            acc_s.at[pl.ds(sub * rpt + i * _ZR, _ZR)])
            return _

        nfull = rpt // _ZR
        rem = rpt - nfull * _ZR
        lax.fori_loop(0, nfull, zacc, 0)
        pltpu.sync_copy(zero_h.at[pl.ds(0, rem)],
                        acc_s.at[pl.ds(sub * rpt + nfull * _ZR, rem)])

        plsc.subcore_barrier()

        def load(cid, b):
            e0 = cid * _CH
            return (pltpu.async_copy(pay_h.at[pl.ds(e0, _CH)], pay_v.at[b],
                                     sem),
                    pltpu.async_copy(dst_h.at[pl.ds(e0, _CH)], dst_v.at[b],
                                     sem))

        def drain(cid, b):
            e0 = cid * _CH
            pltpu.make_async_copy(pay_h.at[pl.ds(e0, _CH)], pay_v.at[b],
                                  sem).wait()
            pltpu.make_async_copy(dst_h.at[pl.ds(e0, _CH)], dst_v.at[b],
                                  sem).wait()

        def scat(b):
            for i in range(_CH // 16):
                v = dst_v[b, pl.ds(i * 16, 16)] - base
                ok = (v >= 0) & (v < _HALF)
                loc_v[pl.ds(i * 16, 16)] = jnp.where(
                    ok, v, jnp.full((16,), _HALF, jnp.int32))
            pltpu.sync_copy(pay_v.at[b], acc_s.at[loc_v], add=True)

        c_base = sub * _CPT
        load(c_base, 0)

        def main(g2, _):
            g = 2 * g2
            drain(c_base + g, 0)
            load(c_base + g + 1, 1)
            scat(0)
            drain(c_base + g + 1, 1)
            load(jnp.minimum(g + 2, _CPT - 1) + c_base, 0)
            scat(1)
            return _

        lax.fori_loop(0, _CPT // 2, main, 0)
        drain(c_base + _CPT - 1, 0)

        @pl.when(sub < n_chunks - 16 * _CPT)
        def _extra():
            cid = 16 * _CPT + sub
            load(cid, 0)
            drain(cid, 0)
            scat(0)

        plsc.subcore_barrier()

        wrows = _HALF // 16

        def wout(i, _):
            r0 = sub * wrows + i * _ZR
            pltpu.sync_copy(acc_s.at[pl.ds(r0, _ZR)],
                            out_h.at[pl.ds(base + r0, _ZR)])
            return _

        nw = wrows // _ZR
        wrem = wrows - nw * _ZR
        lax.fori_loop(0, nw, wout, 0)
        pltpu.sync_copy(acc_s.at[pl.ds(sub * wrows + nw * _ZR, wrem)],
                        out_h.at[pl.ds(base + sub * wrows + nw * _ZR, wrem)])

        @pl.when(sub == 0)
        def _last8():
            pltpu.sync_copy(acc_s.at[pl.ds(16 * wrows, 8)],
                            out_h.at[pl.ds(base + 16 * wrows, 8)])

    return k(pay, dst, zeros)


def _scatter_sum(payload, dst, zeros):
    return _sc_scatter(payload, dst, zeros)


def kernel(x, edge_index, edge_attr, params):
    p = params
    src = edge_index[0]
    dst = edge_index[1]
    zeros = jnp.zeros((_ZR, PAY), jnp.float32)
    x0 = _embed(x, p['node_W'], p['node_b'])

    g0 = _gather(x0, src)
    r0 = _edge_stage(g0, edge_attr, p['edge_W'], p['edge_b'],
                     p['a1W_0'], p['a1b_0'])
    acc0 = _scatter_sum(r0, dst, zeros)
    x1, t1 = _node_stage0(acc0, x0, p)

    g1 = _gather(t1, src)
    r1 = _edge_stage(g1, edge_attr, p['edge_W'], p['edge_b'],
                     p['a1W_1'], p['a1b_1'])
    acc1 = _scatter_sum(r1, dst, zeros)
    return _node_stage1(acc1, t1, x1, p)
